# baseline (speedup 1.0000x reference)
import functools
import numpy as np
import jax
import jax.numpy as jnp
from jax import lax
from jax.experimental import pallas as pl
from jax.experimental.pallas import tpu as pltpu
from jax.experimental.pallas import tpu_sc as plsc

_TRIU_I, _TRIU_J = np.triu_indices(8, k=1)
_NTILES = 32


def _bcast(vec, lane):
    idx = jnp.full((16, 1), lane, jnp.int32)
    dn = lax.GatherDimensionNumbers(
        offset_dims=(), collapsed_slice_dims=(0,), start_index_map=(0,))
    return lax.gather(vec, idx, dn, slice_sizes=(1,),
                      mode=lax.GatherScatterMode.PROMISE_IN_BOUNDS)


def _sc_body(rows, D, xflat, ints_hbm, coef_hbm, out_hbm,
             idx_v, vals_v, outbuf, ints_v, coef_v, sem):
    wid = lax.axis_index("s") * 2 + lax.axis_index("c")
    base = wid * rows
    pltpu.sync_copy(ints_hbm, ints_v)
    pltpu.sync_copy(coef_hbm, coef_v)
    iota = lax.broadcasted_iota(jnp.int32, (16,), 0)
    svec = ints_v[pl.ds(0, 16)]
    sj = [_bcast(svec, j) for j in range(8)]

    for i in range(_NTILES):
        j = i // 4
        for g in range(8):
            q = (i % 4) * 128 + g * 16
            rowv = base + q + iota
            idx_v[i, pl.ds(g * 16, 16)] = rowv * D + sj[j]

    copies = []
    for i in range(_NTILES):
        copies.append(pltpu.async_copy(
            xflat.at[idx_v.at[i]], vals_v.at[pl.ds(i * 128, 128)], sem))
    for c_ in copies:
        c_.wait()

    f3 = [ints_v[pl.ds(16 * (1 + k), 16)] for k in range(3)]
    f4 = [ints_v[pl.ds(16 * (4 + k), 16)] for k in range(4)]
    f5 = [ints_v[pl.ds(16 * (8 + k), 16)] for k in range(5)]
    cv = [coef_v[pl.ds(16 * k, 16)] for k in range(4)]

    def cb(p):
        return _bcast(cv[p // 16], p % 16)

    nrows = jnp.int32(rows)

    def chunk(c, carry):
        col = c * 16 + iota
        v = [vals_v[pl.ds(j * rows + c * 16, 16)] for j in range(8)]
        acc = cb(0) * v[0]
        for j in range(1, 8):
            acc = acc + cb(j) * v[j]
        for t in range(28):
            acc = acc + cb(8 + t) * (v[_TRIU_I[t]] * v[_TRIU_J[t]])
        for m in range(12):
            g = [plsc.load_gather(vals_v, [_bcast(f3[k], m) * nrows + col])
                 for k in range(3)]
            acc = acc + cb(36 + m) * (g[0] * g[1] * g[2])
        for m in range(8):
            g = [plsc.load_gather(vals_v, [_bcast(f4[k], m) * nrows + col])
                 for k in range(4)]
            acc = acc + cb(48 + m) * (g[0] * g[1] * (g[2] * g[3]))
        for m in range(4):
            g = [plsc.load_gather(vals_v, [_bcast(f5[k], m) * nrows + col])
                 for k in range(5)]
            acc = acc + cb(56 + m) * (g[0] * g[1] * (g[2] * g[3]) * g[4])
        outbuf[pl.ds(c * 16, 16)] = acc
        return carry

    lax.fori_loop(0, rows // 16, chunk, 0)
    pltpu.sync_copy(outbuf, out_hbm.at[pl.ds(base, rows)])


def _pad16(v):
    return jnp.pad(v, (0, 16 - v.shape[0]))


def kernel(x, a, b, c3, c4, c5, S, idx3, idx4, idx5):
    B, D = x.shape
    rows = B // _NTILES
    s32 = S.astype(jnp.int32)
    i3 = idx3.astype(jnp.int32)
    i4 = idx4.astype(jnp.int32)
    i5 = idx5.astype(jnp.int32)
    ints = jnp.concatenate(
        [_pad16(s32)]
        + [_pad16(i3[:, k]) for k in range(3)]
        + [_pad16(i4[:, k]) for k in range(4)]
        + [_pad16(i5[:, k]) for k in range(5)])
    ti = jnp.asarray(_TRIU_I, jnp.int32)
    tj = jnp.asarray(_TRIU_J, jnp.int32)
    coef = jnp.concatenate(
        [a, b[ti, tj], c3, c4, c5, jnp.zeros((4,), jnp.float32)])
    xflat = x.reshape(-1)

    mesh = plsc.VectorSubcoreMesh(core_axis_name="c", subcore_axis_name="s")
    run = functools.partial(
        pl.kernel,
        mesh=mesh,
        compiler_params=pltpu.CompilerParams(needs_layout_passes=False),
        out_type=jax.ShapeDtypeStruct((B,), jnp.float32),
        scratch_types=[
            pltpu.VMEM((_NTILES, 128), jnp.int32),
            pltpu.VMEM((8 * rows,), jnp.float32),
            pltpu.VMEM((rows,), jnp.float32),
            pltpu.VMEM((208,), jnp.int32),
            pltpu.VMEM((64,), jnp.float32),
            pltpu.SemaphoreType.DMA,
        ],
    )(functools.partial(_sc_body, rows, D))
    return run(xflat, ints, coef)


# speedup vs baseline: 7.9352x; 7.9352x over previous
import numpy as np
import jax
import jax.numpy as jnp
from jax.experimental import pallas as pl
from jax.experimental.pallas import tpu as pltpu

_R = 1024
_TRIU_I, _TRIU_J = np.triu_indices(8, k=1)


def _poly_body(s_ref, *refs):
    xblks = refs[:8]
    e_ref, g_ref, out_ref = refs[8], refs[9], refs[10]
    xsa = jnp.zeros((_R, 16), dtype=jnp.float32)
    for j in range(8):
        xsa = xsa + jnp.dot(xblks[j][...], e_ref[j],
                            preferred_element_type=jnp.float32)
    lane16 = jax.lax.broadcasted_iota(jnp.int32, (_R, 16), 1)
    xsa = xsa + jnp.where(lane16 == 8, 1.0, 0.0)
    v = jnp.dot(xsa, g_ref[...], preferred_element_type=jnp.float32)
    p = (v[:, 0:128] * v[:, 128:256] * v[:, 256:384]
         * v[:, 384:512] * v[:, 512:640])
    out_ref[...] = jnp.sum(p, axis=1, keepdims=True)


def kernel(x, a, b, c3, c4, c5, S, idx3, idx4, idx5):
    B, D = x.shape
    s32 = S.astype(jnp.int32)
    i3 = idx3.astype(jnp.int32)
    i4 = idx4.astype(jnp.int32)
    i5 = idx5.astype(jnp.int32)

    lj = s32 % 128
    E = ((jnp.arange(128, dtype=jnp.int32)[None, :, None] == lj[:, None, None])
         & (jnp.arange(16, dtype=jnp.int32)[None, None, :]
            == jnp.arange(8, dtype=jnp.int32)[:, None, None])
         ).astype(jnp.float32)

    ones8 = jnp.full((8,), 8, jnp.int32)
    ones28 = jnp.full((28,), 8, jnp.int32)
    ones12 = jnp.full((12,), 8, jnp.int32)
    ones4 = jnp.full((4,), 8, jnp.int32)
    ti = jnp.asarray(_TRIU_I, jnp.int32)
    tj = jnp.asarray(_TRIU_J, jnp.int32)
    lin = jnp.arange(8, dtype=jnp.int32)
    feat = [
        jnp.concatenate([lin, ti, i3[:, 0], i4[:, 0], i5[:, 0]]),
        jnp.concatenate([ones8, tj, i3[:, 1], i4[:, 1], i5[:, 1]]),
        jnp.concatenate([ones8, ones28, i3[:, 2], i4[:, 2], i5[:, 2]]),
        jnp.concatenate([ones8, ones28, ones12, i4[:, 3], i5[:, 3]]),
        jnp.concatenate([ones8, ones28, ones12, jnp.full((8,), 8, jnp.int32),
                         i5[:, 4]]),
    ]
    coef = jnp.concatenate([a, b[ti, tj], c3, c4, c5])

    gtiles = []
    ar16 = jnp.arange(16, dtype=jnp.int32)
    for k in range(5):
        oh = (feat[k][:, None] == ar16[None, :]).astype(jnp.float32)
        if k == 0:
            oh = oh * coef[:, None]
        gk = jnp.pad(oh.T, ((0, 0), (0, 68)))
        gtiles.append(gk)
    G = jnp.concatenate(gtiles, axis=1)

    nb = B // _R
    grid_spec = pltpu.PrefetchScalarGridSpec(
        num_scalar_prefetch=1,
        grid=(nb,),
        in_specs=(
            [pl.BlockSpec((_R, 128), (lambda i, s, j=j: (i, s[j] // 128)))
             for j in range(8)]
            + [pl.BlockSpec((8, 128, 16), lambda i, s: (0, 0, 0)),
               pl.BlockSpec((16, 640), lambda i, s: (0, 0))]
        ),
        out_specs=pl.BlockSpec((_R, 1), lambda i, s: (i, 0)),
    )
    out = pl.pallas_call(
        _poly_body,
        grid_spec=grid_spec,
        out_shape=jax.ShapeDtypeStruct((B, 1), jnp.float32),
        compiler_params=pltpu.CompilerParams(
            dimension_semantics=("arbitrary",),
        ),
    )(s32, x, x, x, x, x, x, x, x, E, G)
    return out.reshape(B)


# speedup vs baseline: 8.6225x; 1.0866x over previous
import numpy as np
import jax
import jax.numpy as jnp
from jax.experimental import pallas as pl
from jax.experimental.pallas import tpu as pltpu

_R = 2048
_TRIU_I, _TRIU_J = np.triu_indices(8, k=1)


def _poly_body(s_ref, *refs):
    xblks = refs[:8]
    e_ref, g_ref, out_ref = refs[8], refs[9], refs[10]
    xsa = jnp.zeros((_R, 16), dtype=jnp.float32)
    for j in range(8):
        xsa = xsa + jnp.dot(xblks[j][...], e_ref[j],
                            preferred_element_type=jnp.float32)
    lane16 = jax.lax.broadcasted_iota(jnp.int32, (_R, 16), 1)
    xsa = xsa + jnp.where(lane16 == 8, 1.0, 0.0)
    v = jnp.dot(xsa, g_ref[...], preferred_element_type=jnp.float32)
    p = (v[:, 0:128] * v[:, 128:256] * v[:, 256:384]
         * v[:, 384:512] * v[:, 512:640])
    out_ref[...] = jnp.sum(p, axis=1, keepdims=True)


def kernel(x, a, b, c3, c4, c5, S, idx3, idx4, idx5):
    B, D = x.shape
    s32 = S.astype(jnp.int32)
    i3 = idx3.astype(jnp.int32)
    i4 = idx4.astype(jnp.int32)
    i5 = idx5.astype(jnp.int32)

    lj = s32 % 128
    E = ((jnp.arange(128, dtype=jnp.int32)[None, :, None] == lj[:, None, None])
         & (jnp.arange(16, dtype=jnp.int32)[None, None, :]
            == jnp.arange(8, dtype=jnp.int32)[:, None, None])
         ).astype(jnp.float32)

    ones8 = jnp.full((8,), 8, jnp.int32)
    ones28 = jnp.full((28,), 8, jnp.int32)
    ones12 = jnp.full((12,), 8, jnp.int32)
    ones4 = jnp.full((4,), 8, jnp.int32)
    ti = jnp.asarray(_TRIU_I, jnp.int32)
    tj = jnp.asarray(_TRIU_J, jnp.int32)
    lin = jnp.arange(8, dtype=jnp.int32)
    feat = [
        jnp.concatenate([lin, ti, i3[:, 0], i4[:, 0], i5[:, 0]]),
        jnp.concatenate([ones8, tj, i3[:, 1], i4[:, 1], i5[:, 1]]),
        jnp.concatenate([ones8, ones28, i3[:, 2], i4[:, 2], i5[:, 2]]),
        jnp.concatenate([ones8, ones28, ones12, i4[:, 3], i5[:, 3]]),
        jnp.concatenate([ones8, ones28, ones12, jnp.full((8,), 8, jnp.int32),
                         i5[:, 4]]),
    ]
    coef = jnp.concatenate([a, b[ti, tj], c3, c4, c5])

    gtiles = []
    ar16 = jnp.arange(16, dtype=jnp.int32)
    for k in range(5):
        oh = (feat[k][:, None] == ar16[None, :]).astype(jnp.float32)
        if k == 0:
            oh = oh * coef[:, None]
        gk = jnp.pad(oh.T, ((0, 0), (0, 68)))
        gtiles.append(gk)
    G = jnp.concatenate(gtiles, axis=1)

    nb = B // _R
    grid_spec = pltpu.PrefetchScalarGridSpec(
        num_scalar_prefetch=1,
        grid=(nb,),
        in_specs=(
            [pl.BlockSpec((_R, 128), (lambda i, s, j=j: (i, s[j] // 128)))
             for j in range(8)]
            + [pl.BlockSpec((8, 128, 16), lambda i, s: (0, 0, 0)),
               pl.BlockSpec((16, 640), lambda i, s: (0, 0))]
        ),
        out_specs=pl.BlockSpec((_R, 1), lambda i, s: (i, 0)),
    )
    out = pl.pallas_call(
        _poly_body,
        grid_spec=grid_spec,
        out_shape=jax.ShapeDtypeStruct((B, 1), jnp.float32),
        compiler_params=pltpu.CompilerParams(
            dimension_semantics=("arbitrary",),
        ),
    )(s32, x, x, x, x, x, x, x, x, E, G)
    return out.reshape(B)


# speedup vs baseline: 8.6470x; 1.0028x over previous
import numpy as np
import jax
import jax.numpy as jnp
from jax.experimental import pallas as pl
from jax.experimental.pallas import tpu as pltpu

_R = 4096
_TRIU_I, _TRIU_J = np.triu_indices(8, k=1)


def _poly_body(s_ref, *refs):
    xblks = refs[:8]
    e_ref, g_ref, out_ref = refs[8], refs[9], refs[10]
    xsa = jnp.zeros((_R, 16), dtype=jnp.float32)
    for j in range(8):
        xsa = xsa + jnp.dot(xblks[j][...], e_ref[j],
                            preferred_element_type=jnp.float32)
    lane16 = jax.lax.broadcasted_iota(jnp.int32, (_R, 16), 1)
    xsa = xsa + jnp.where(lane16 == 8, 1.0, 0.0)
    v = jnp.dot(xsa, g_ref[...], preferred_element_type=jnp.float32)
    p = (v[:, 0:128] * v[:, 128:256] * v[:, 256:384]
         * v[:, 384:512] * v[:, 512:640])
    out_ref[...] = jnp.sum(p, axis=1, keepdims=True)


def kernel(x, a, b, c3, c4, c5, S, idx3, idx4, idx5):
    B, D = x.shape
    s32 = S.astype(jnp.int32)
    i3 = idx3.astype(jnp.int32)
    i4 = idx4.astype(jnp.int32)
    i5 = idx5.astype(jnp.int32)

    lj = s32 % 128
    E = ((jnp.arange(128, dtype=jnp.int32)[None, :, None] == lj[:, None, None])
         & (jnp.arange(16, dtype=jnp.int32)[None, None, :]
            == jnp.arange(8, dtype=jnp.int32)[:, None, None])
         ).astype(jnp.float32)

    ones8 = jnp.full((8,), 8, jnp.int32)
    ones28 = jnp.full((28,), 8, jnp.int32)
    ones12 = jnp.full((12,), 8, jnp.int32)
    ones4 = jnp.full((4,), 8, jnp.int32)
    ti = jnp.asarray(_TRIU_I, jnp.int32)
    tj = jnp.asarray(_TRIU_J, jnp.int32)
    lin = jnp.arange(8, dtype=jnp.int32)
    feat = [
        jnp.concatenate([lin, ti, i3[:, 0], i4[:, 0], i5[:, 0]]),
        jnp.concatenate([ones8, tj, i3[:, 1], i4[:, 1], i5[:, 1]]),
        jnp.concatenate([ones8, ones28, i3[:, 2], i4[:, 2], i5[:, 2]]),
        jnp.concatenate([ones8, ones28, ones12, i4[:, 3], i5[:, 3]]),
        jnp.concatenate([ones8, ones28, ones12, jnp.full((8,), 8, jnp.int32),
                         i5[:, 4]]),
    ]
    coef = jnp.concatenate([a, b[ti, tj], c3, c4, c5])

    gtiles = []
    ar16 = jnp.arange(16, dtype=jnp.int32)
    for k in range(5):
        oh = (feat[k][:, None] == ar16[None, :]).astype(jnp.float32)
        if k == 0:
            oh = oh * coef[:, None]
        gk = jnp.pad(oh.T, ((0, 0), (0, 68)))
        gtiles.append(gk)
    G = jnp.concatenate(gtiles, axis=1)

    nb = B // _R
    grid_spec = pltpu.PrefetchScalarGridSpec(
        num_scalar_prefetch=1,
        grid=(nb,),
        in_specs=(
            [pl.BlockSpec((_R, 128), (lambda i, s, j=j: (i, s[j] // 128)))
             for j in range(8)]
            + [pl.BlockSpec((8, 128, 16), lambda i, s: (0, 0, 0)),
               pl.BlockSpec((16, 640), lambda i, s: (0, 0))]
        ),
        out_specs=pl.BlockSpec((_R, 1), lambda i, s: (i, 0)),
    )
    out = pl.pallas_call(
        _poly_body,
        grid_spec=grid_spec,
        out_shape=jax.ShapeDtypeStruct((B, 1), jnp.float32),
        compiler_params=pltpu.CompilerParams(
            dimension_semantics=("arbitrary",),
        ),
    )(s32, x, x, x, x, x, x, x, x, E, G)
    return out.reshape(B)
